# baseline (speedup 1.0000x reference)
import functools

import numpy as np
import jax
import jax.numpy as jnp
from jax import lax
from jax.experimental import pallas as pl
from jax.experimental.pallas import tpu as pltpu
from jax.experimental.pallas import tpu_sc as plsc

VOCAB = 100000
EMBED = 128
WINDOW = 2048
BATCH = 4
SEQ = 2048

SCALE = float(np.sqrt(float(EMBED)))

NUM_CORES = 2
NUM_SUBCORES = 16
NW = NUM_CORES * NUM_SUBCORES
PPW = SEQ // NW
LANES = 16
NVEC = EMBED // LANES
HALF = EMBED // 2


def _pe_tables() -> np.ndarray:
    rates = 1.0 / 10000 ** (np.arange(HALF) / HALF)
    r = np.concatenate([rates, rates])
    p0 = (np.arange(NW) * PPW)[:, np.newaxis]
    tab = np.concatenate(
        [np.sin(p0 * r), np.cos(p0 * r), np.cos(r)[np.newaxis], np.sin(r)[np.newaxis]],
        axis=0,
    )
    return tab.astype(np.float32)


_TAB_NP = _pe_tables()

_MESH = plsc.VectorSubcoreMesh(
    core_axis_name="c", subcore_axis_name="s",
    num_cores=NUM_CORES, num_subcores=NUM_SUBCORES,
)


@functools.partial(
    pl.kernel,
    out_type=jax.ShapeDtypeStruct((BATCH, SEQ, EMBED), jnp.float32),
    mesh=_MESH,
    scratch_types=[
        pltpu.VMEM((BATCH, PPW), jnp.int32),
        pltpu.VMEM((BATCH, PPW, EMBED), jnp.float32),
        pltpu.VMEM((PPW, EMBED), jnp.float32),
        pltpu.VMEM((4, EMBED), jnp.float32),
        pltpu.SemaphoreType.DMA,
        pltpu.SemaphoreType.DMA,
    ],
)
def _sc_embed(x_hbm, table_hbm, tab_hbm, out_hbm, idx_v, rows_v, pe_v, ab_v,
              sem, osem):
    wid = lax.axis_index("s") * NUM_CORES + lax.axis_index("c")
    p0 = wid * PPW

    idx_cps = [
        pltpu.async_copy(x_hbm.at[b, pl.ds(p0, PPW)], idx_v.at[b], sem)
        for b in range(BATCH)
    ]
    pltpu.sync_copy(tab_hbm.at[wid], ab_v.at[0])
    pltpu.sync_copy(tab_hbm.at[NW + wid], ab_v.at[1])
    pltpu.sync_copy(tab_hbm.at[2 * NW], ab_v.at[2])
    pltpu.sync_copy(tab_hbm.at[2 * NW + 1], ab_v.at[3])
    for cp in idx_cps:
        cp.wait()

    gathers = [
        pltpu.async_copy(table_hbm.at[idx_v.at[b]], rows_v.at[b], sem)
        for b in range(BATCH)
    ]

    svec = [ab_v[0, pl.ds(j * LANES, LANES)] for j in range(NVEC)]
    cvec = [ab_v[1, pl.ds(j * LANES, LANES)] for j in range(NVEC)]
    cr = [ab_v[2, pl.ds(j * LANES, LANES)] for j in range(NVEC)]
    sr = [ab_v[3, pl.ds(j * LANES, LANES)] for j in range(NVEC)]

    def pe_body(q, carry):
        s = carry[:NVEC]
        c = carry[NVEC:]
        for j in range(NVEC):
            sl = pl.ds(j * LANES, LANES)
            pe_v[q, sl] = s[j] if j < NVEC // 2 else c[j]
        s_n = [s[j] * cr[j] + c[j] * sr[j] for j in range(NVEC)]
        c_n = [c[j] * cr[j] - s[j] * sr[j] for j in range(NVEC)]
        return tuple(s_n) + tuple(c_n)

    lax.fori_loop(0, PPW, pe_body, tuple(svec) + tuple(cvec))

    outs = []
    for b in range(BATCH):
        gathers[b].wait()

        def body(q, carry, b=b):
            for j in range(NVEC):
                sl = pl.ds(j * LANES, LANES)
                rows_v[b, q, sl] = rows_v[b, q, sl] * SCALE + pe_v[q, sl]
            return carry

        lax.fori_loop(0, PPW, body, 0)
        outs.append(
            pltpu.async_copy(rows_v.at[b], out_hbm.at[b].at[pl.ds(p0, PPW)], osem)
        )
    for o in outs:
        o.wait()


def kernel(x, table):
    return _sc_embed(x.astype(jnp.int32), table, jnp.asarray(_TAB_NP))


# speedup vs baseline: 1.1250x; 1.1250x over previous
import functools

import numpy as np
import jax
import jax.numpy as jnp
from jax import lax
from jax.experimental import pallas as pl
from jax.experimental.pallas import tpu as pltpu
from jax.experimental.pallas import tpu_sc as plsc

VOCAB = 100000
EMBED = 128
WINDOW = 2048
BATCH = 4
SEQ = 2048

SCALE = float(np.sqrt(float(EMBED)))

NUM_CORES = 2
NUM_SUBCORES = 16
NW = NUM_CORES * NUM_SUBCORES
PPW = SEQ // NW
LANES = 16
NVEC = EMBED // LANES
HALF = EMBED // 2


def _pe_tables() -> np.ndarray:
    rates = 1.0 / 10000 ** (np.arange(HALF) / HALF)
    r = np.concatenate([rates, rates])
    p0 = (np.arange(NW) * PPW)[:, np.newaxis]
    tab = np.stack(
        [
            np.sin(p0 * r),
            np.cos(p0 * r),
            np.broadcast_to(np.cos(r), (NW, EMBED)),
            np.broadcast_to(np.sin(r), (NW, EMBED)),
        ],
        axis=1,
    )
    return np.ascontiguousarray(tab).astype(np.float32)


_TAB_NP = _pe_tables()

_MESH = plsc.VectorSubcoreMesh(
    core_axis_name="c", subcore_axis_name="s",
    num_cores=NUM_CORES, num_subcores=NUM_SUBCORES,
)


@functools.partial(
    pl.kernel,
    out_type=jax.ShapeDtypeStruct((BATCH, SEQ, EMBED), jnp.float32),
    mesh=_MESH,
    scratch_types=[
        pltpu.VMEM((BATCH, PPW), jnp.int32),
        pltpu.VMEM((BATCH, PPW, EMBED), jnp.float32),
        pltpu.VMEM((PPW, EMBED), jnp.float32),
        pltpu.VMEM((4, EMBED), jnp.float32),
        pltpu.SemaphoreType.DMA,
        pltpu.SemaphoreType.DMA,
    ],
)
def _sc_embed(x_hbm, table_hbm, tab_hbm, out_hbm, idx_v, rows_v, pe_v, ab_v,
              sem, osem):
    wid = lax.axis_index("s") * NUM_CORES + lax.axis_index("c")
    p0 = wid * PPW

    tab_cp = pltpu.async_copy(tab_hbm.at[wid], ab_v, osem)
    idx_cps = [
        pltpu.async_copy(x_hbm.at[b, pl.ds(p0, PPW)], idx_v.at[b], sem)
        for b in range(BATCH)
    ]
    gathers = []
    for b in range(BATCH):
        idx_cps[b].wait()
        gathers.append(
            pltpu.async_copy(table_hbm.at[idx_v.at[b]], rows_v.at[b], sem)
        )
    tab_cp.wait()

    svec = [ab_v[0, pl.ds(j * LANES, LANES)] for j in range(NVEC)]
    cvec = [ab_v[1, pl.ds(j * LANES, LANES)] for j in range(NVEC)]
    cr = [ab_v[2, pl.ds(j * LANES, LANES)] for j in range(NVEC)]
    sr = [ab_v[3, pl.ds(j * LANES, LANES)] for j in range(NVEC)]

    def pe_body(q, carry):
        s = carry[:NVEC]
        c = carry[NVEC:]
        for j in range(NVEC):
            sl = pl.ds(j * LANES, LANES)
            pe_v[q, sl] = s[j] if j < NVEC // 2 else c[j]
        s_n = [s[j] * cr[j] + c[j] * sr[j] for j in range(NVEC)]
        c_n = [c[j] * cr[j] - s[j] * sr[j] for j in range(NVEC)]
        return tuple(s_n) + tuple(c_n)

    lax.fori_loop(0, PPW, pe_body, tuple(svec) + tuple(cvec))

    outs = []
    for b in range(BATCH):
        gathers[b].wait()

        def body(q, carry, b=b):
            for j in range(NVEC):
                sl = pl.ds(j * LANES, LANES)
                rows_v[b, q, sl] = rows_v[b, q, sl] * SCALE + pe_v[q, sl]
            return carry

        lax.fori_loop(0, PPW, body, 0)
        outs.append(
            pltpu.async_copy(rows_v.at[b], out_hbm.at[b].at[pl.ds(p0, PPW)], osem)
        )
    for o in outs:
        o.wait()


def kernel(x, table):
    return _sc_embed(x.astype(jnp.int32), table, jnp.asarray(_TAB_NP))
